# baseline (speedup 1.0000x reference)
import functools

import jax
import jax.numpy as jnp
from jax import lax
from jax.experimental import pallas as pl
from jax.experimental.pallas import tpu as pltpu
from jax.experimental.pallas import tpu_sc as plsc

T, D, HR, E, DFF, C = 2048, 1024, 512, 8, 4096, 1024
BM_R = 512
BM = 256
BF = 512
F = DFF // BF
M_PAD = 2 * T + E * BM
MAX_ROWS = T
NC, NS, NW = 2, 16, 32
TPW = T // NW
HALF = TPW // 2


def _router_body(x_ref, rw1_ref, rb1_ref, rw2_ref, rb2_ref, rw3_ref, rb3_ref,
                 probs_ref, w1_ref, w2_ref, div_ref, s1_ref, s2_ref, cnt_ref,
                 off_ref, i1sc, i2sc):
    m = pl.program_id(0)
    h1 = jnp.maximum(
        jnp.dot(x_ref[...], rw1_ref[...], preferred_element_type=jnp.float32)
        + rb1_ref[...], 0.0)
    h2 = jnp.maximum(
        jnp.dot(h1, rw2_ref[...], preferred_element_type=jnp.float32)
        + rb2_ref[...], 0.0)
    scores = (jnp.dot(h2, rw3_ref[...], preferred_element_type=jnp.float32)
              + rb3_ref[...])
    mx = jnp.max(scores, axis=1, keepdims=True)
    ex = jnp.exp(scores - mx)
    probs = ex / jnp.sum(ex, axis=1, keepdims=True)
    probs_ref[...] = probs
    iota = jax.lax.broadcasted_iota(jnp.int32, probs.shape, 1)
    p1 = jnp.max(probs, axis=1, keepdims=True)
    i1 = jnp.min(jnp.where(probs == p1, iota, E), axis=1, keepdims=True)
    masked = jnp.where(iota == i1, -jnp.inf, probs)
    p2 = jnp.max(masked, axis=1, keepdims=True)
    i2 = jnp.min(jnp.where(masked == p2, iota, E), axis=1, keepdims=True)
    w1_ref[...] = jnp.broadcast_to(p1 * 0.5, (BM_R, 16))
    w2_ref[...] = jnp.broadcast_to(p2 * 0.5, (BM_R, 16))
    i1sc[pl.ds(pl.multiple_of(m * BM_R, BM_R), BM_R), :] = i1
    i2sc[pl.ds(pl.multiple_of(m * BM_R, BM_R), BM_R), :] = i2
    dv = jnp.sum(scores * scores)

    @pl.when(m == 0)
    def _():
        div_ref[0, 0] = dv

    @pl.when(m != 0)
    def _():
        div_ref[0, 0] += dv

    @pl.when(m == T // BM_R - 1)
    def _():
        ee = jax.lax.broadcasted_iota(jnp.int32, (T, E), 1)
        a1 = (i1sc[...] == ee).astype(jnp.float32)
        a2 = (i2sc[...] == ee).astype(jnp.float32)
        a = a1 + a2
        CH = 512
        ri = jax.lax.broadcasted_iota(jnp.int32, (CH, CH), 0)
        ci = jax.lax.broadcasted_iota(jnp.int32, (CH, CH), 1)
        tri = (ci <= ri).astype(jnp.float32)
        base = jnp.zeros((1, E), jnp.float32)
        chunks = []
        for k in range(T // CH):
            ak = a[k * CH:(k + 1) * CH]
            incl = jnp.dot(tri, ak, preferred_element_type=jnp.float32)
            chunks.append(incl - ak + base)
            base = base + incl[CH - 1:CH, :]
        excl = jnp.concatenate(chunks, axis=0)
        counts = base
        ntiles = jnp.floor((counts + (BM - 1)) * (1.0 / BM))
        padded = ntiles * BM
        ii = jax.lax.broadcasted_iota(jnp.int32, (E, E), 0)
        jj = jax.lax.broadcasted_iota(jnp.int32, (E, E), 1)
        pmat = jnp.broadcast_to(padded, (E, E))
        off_col = jnp.sum(jnp.where(jj < ii, pmat, 0.0), axis=1,
                          keepdims=True)
        cnt_col = jnp.sum(jnp.where(jj == ii,
                                    jnp.broadcast_to(counts, (E, E)), 0.0),
                          axis=1, keepdims=True)
        off_ref[...] = off_col.astype(jnp.int32)
        cnt_ref[...] = cnt_col.astype(jnp.int32)
        rank1 = jnp.sum(excl * a1, axis=1, keepdims=True)
        rank2 = jnp.sum(excl * a2, axis=1, keepdims=True)
        off1 = jnp.dot(a1, off_col, preferred_element_type=jnp.float32)
        off2 = jnp.dot(a2, off_col, preferred_element_type=jnp.float32)
        s1_ref[...] = (off1 + rank1).astype(jnp.int32)
        s2_ref[...] = (off2 + rank2).astype(jnp.int32)


def _router_call(x, RW1, Rb1, RW2, Rb2, RW3, Rb3):
    return pl.pallas_call(
        _router_body,
        grid=(T // BM_R,),
        in_specs=[
            pl.BlockSpec((BM_R, D), lambda m: (m, 0)),
            pl.BlockSpec((D, HR), lambda m: (0, 0)),
            pl.BlockSpec((1, HR), lambda m: (0, 0)),
            pl.BlockSpec((HR, HR // 2), lambda m: (0, 0)),
            pl.BlockSpec((1, HR // 2), lambda m: (0, 0)),
            pl.BlockSpec((HR // 2, E), lambda m: (0, 0)),
            pl.BlockSpec((1, E), lambda m: (0, 0)),
        ],
        out_specs=[
            pl.BlockSpec((BM_R, E), lambda m: (m, 0)),
            pl.BlockSpec((BM_R, 16), lambda m: (m, 0)),
            pl.BlockSpec((BM_R, 16), lambda m: (m, 0)),
            pl.BlockSpec(memory_space=pltpu.SMEM),
            pl.BlockSpec((T, 1), lambda m: (0, 0)),
            pl.BlockSpec((T, 1), lambda m: (0, 0)),
            pl.BlockSpec((E, 1), lambda m: (0, 0)),
            pl.BlockSpec((E, 1), lambda m: (0, 0)),
        ],
        out_shape=[
            jax.ShapeDtypeStruct((T, E), jnp.float32),
            jax.ShapeDtypeStruct((T, 16), jnp.float32),
            jax.ShapeDtypeStruct((T, 16), jnp.float32),
            jax.ShapeDtypeStruct((1, 1), jnp.float32),
            jax.ShapeDtypeStruct((T, 1), jnp.int32),
            jax.ShapeDtypeStruct((T, 1), jnp.int32),
            jax.ShapeDtypeStruct((E, 1), jnp.int32),
            jax.ShapeDtypeStruct((E, 1), jnp.int32),
        ],
        scratch_shapes=[
            pltpu.VMEM((T, 1), jnp.int32),
            pltpu.VMEM((T, 1), jnp.int32),
        ],
        compiler_params=pltpu.CompilerParams(
            dimension_semantics=("arbitrary",)),
    )(x, RW1, Rb1.reshape(1, -1), RW2, Rb2.reshape(1, -1), RW3,
      Rb3.reshape(1, -1))


def _dispatch_body(x_hbm, s1_hbm, s2_hbm, xs_hbm, idx1_v, idx2_v, rows_v,
                   sem):
    wid = lax.axis_index("s") * NC + lax.axis_index("c")
    base = wid * TPW
    pltpu.sync_copy(s1_hbm.at[pl.ds(base, TPW)], idx1_v)
    pltpu.sync_copy(s2_hbm.at[pl.ds(base, TPW)], idx2_v)
    pltpu.sync_copy(x_hbm.at[pl.ds(base, TPW)], rows_v)
    cp1 = pltpu.async_copy(rows_v, xs_hbm.at[idx1_v], sem)
    cp2 = pltpu.async_copy(rows_v, xs_hbm.at[idx2_v], sem)
    cp1.wait()
    cp2.wait()


def _dispatch_call(x, s1, s2):
    fn = functools.partial(
        pl.kernel,
        mesh=plsc.VectorSubcoreMesh(core_axis_name="c", subcore_axis_name="s"),
        out_type=jax.ShapeDtypeStruct((M_PAD, D), jnp.float32),
        scratch_types=[
            pltpu.VMEM((TPW,), jnp.int32),
            pltpu.VMEM((TPW,), jnp.int32),
            pltpu.VMEM((TPW, D), jnp.float32),
            pltpu.SemaphoreType.DMA,
        ],
    )(_dispatch_body)
    return fn(x, s1, s2)


def _gmm_body(cnt_ref, off_ref, xs_ref, ew1_ref, eb1_ref, ew2_ref, eb2_ref,
              y_ref, xsc0, xsc1, ysc, sem1, sem2):
    e = pl.program_id(0)
    f = pl.program_id(1)
    cnt = cnt_ref[e, 0]
    off = off_ref[e, 0]
    nt = (cnt + (BM - 1)) // BM

    def nt_of(ei):
        return (cnt_ref[ei, 0] + (BM - 1)) // BM

    def x_copy(ei, i, buf):
        return pltpu.make_async_copy(
            xs_ref.at[pl.ds(pl.multiple_of(off_ref[ei, 0] + i * BM, BM), BM)],
            buf.at[pl.ds(i * BM, BM)], sem1)

    def y_copy(ei, i):
        return pltpu.make_async_copy(
            ysc.at[pl.ds(i * BM, BM)],
            y_ref.at[pl.ds(pl.multiple_of(off_ref[ei, 0] + i * BM, BM), BM)],
            sem2)

    @pl.when(f == 0)
    def _():
        @pl.when(e == 0)
        def _():
            lax.fori_loop(
                0, nt, lambda i, c: (x_copy(0, i, xsc0).start(), c)[1], 0)

        @pl.when(e % 2 == 0)
        def _():
            lax.fori_loop(
                0, nt, lambda i, c: (x_copy(e, i, xsc0).wait(), c)[1], 0)

        @pl.when(e % 2 == 1)
        def _():
            lax.fori_loop(
                0, nt, lambda i, c: (x_copy(e, i, xsc1).wait(), c)[1], 0)

        @pl.when(e > 0)
        def _():
            lax.fori_loop(
                0, nt_of(e - 1),
                lambda i, c: (y_copy(e - 1, i).wait(), c)[1], 0)

    @pl.when((f == F - 1) & (e + 1 < E) & (e % 2 == 0))
    def _():
        lax.fori_loop(
            0, nt_of(e + 1),
            lambda i, c: (x_copy(e + 1, i, xsc1).start(), c)[1], 0)

    @pl.when((f == F - 1) & (e + 1 < E) & (e % 2 == 1))
    def _():
        lax.fori_loop(
            0, nt_of(e + 1),
            lambda i, c: (x_copy(e + 1, i, xsc0).start(), c)[1], 0)

    ew1 = ew1_ref[0]
    eb1 = eb1_ref[0]
    ew2 = ew2_ref[0]
    eb2 = eb2_ref[0]

    def tile_loop(xbuf):
        def tile(i, carry):
            sl = pl.ds(i * BM, BM)
            hh = jnp.maximum(
                jnp.dot(xbuf[sl, :], ew1, preferred_element_type=jnp.float32)
                + eb1, 0.0)
            contrib = jnp.dot(hh, ew2, preferred_element_type=jnp.float32)

            @pl.when(f == 0)
            def _():
                ysc[sl, :] = contrib + eb2

            @pl.when(f != 0)
            def _():
                ysc[sl, :] = ysc[sl, :] + contrib

            return carry

        lax.fori_loop(0, nt, tile, 0)

    @pl.when(e % 2 == 0)
    def _():
        tile_loop(xsc0)

    @pl.when(e % 2 == 1)
    def _():
        tile_loop(xsc1)

    @pl.when(f == F - 1)
    def _():
        lax.fori_loop(0, nt, lambda i, c: (y_copy(e, i).start(), c)[1], 0)

        @pl.when(e == E - 1)
        def _():
            lax.fori_loop(0, nt, lambda i, c: (y_copy(e, i).wait(), c)[1], 0)


def _gmm_call(cnt, off, xs, EW1, Eb1, EW2, Eb2):
    return pl.pallas_call(
        _gmm_body,
        grid=(E, F),
        in_specs=[
            pl.BlockSpec(memory_space=pltpu.SMEM),
            pl.BlockSpec(memory_space=pltpu.SMEM),
            pl.BlockSpec(memory_space=pl.ANY),
            pl.BlockSpec((1, D, BF), lambda e, f: (e, 0, f)),
            pl.BlockSpec((1, 1, BF), lambda e, f: (e, 0, f)),
            pl.BlockSpec((1, BF, C), lambda e, f: (e, f, 0)),
            pl.BlockSpec((1, 1, C), lambda e, f: (e, 0, 0)),
        ],
        out_specs=pl.BlockSpec(memory_space=pl.ANY),
        out_shape=jax.ShapeDtypeStruct((M_PAD, C), jnp.float32),
        scratch_shapes=[
            pltpu.VMEM((MAX_ROWS, D), jnp.float32),
            pltpu.VMEM((MAX_ROWS, D), jnp.float32),
            pltpu.VMEM((MAX_ROWS, C), jnp.float32),
            pltpu.SemaphoreType.DMA,
            pltpu.SemaphoreType.DMA,
        ],
        compiler_params=pltpu.CompilerParams(
            dimension_semantics=("arbitrary", "arbitrary")),
    )(cnt, off, xs, EW1, Eb1.reshape(E, 1, DFF), EW2, Eb2.reshape(E, 1, C))


CHT = 16
NH = TPW // CHT


def _combine_body(y_hbm, s1_hbm, s2_hbm, w1_hbm, w2_hbm, out_hbm,
                  ia1, ia2, ib1, ib2, y1a, y2a, y1b, y2b, w1_v, w2_v, o_v,
                  sema, semb):
    wid = lax.axis_index("s") * NC + lax.axis_index("c")
    tbase = wid * TPW
    pltpu.sync_copy(w1_hbm.at[pl.ds(tbase, TPW)], w1_v)
    pltpu.sync_copy(w2_hbm.at[pl.ds(tbase, TPW)], w2_v)
    pltpu.sync_copy(s1_hbm.at[pl.ds(tbase, CHT)], ia1)
    pltpu.sync_copy(s2_hbm.at[pl.ds(tbase, CHT)], ia2)
    pending = [pltpu.async_copy(y_hbm.at[ia1], y1a, sema),
               pltpu.async_copy(y_hbm.at[ia2], y2a, sema)]
    for h in range(NH):
        base = tbase + h * CHT
        even = (h % 2 == 0)
        if h + 1 < NH:
            nbase = base + CHT
            (ni1, ni2, ny1, ny2, nsem) = (
                (ib1, ib2, y1b, y2b, semb) if even else
                (ia1, ia2, y1a, y2a, sema))
            pltpu.sync_copy(s1_hbm.at[pl.ds(nbase, CHT)], ni1)
            pltpu.sync_copy(s2_hbm.at[pl.ds(nbase, CHT)], ni2)
            nxt = [pltpu.async_copy(y_hbm.at[ni1], ny1, nsem),
                   pltpu.async_copy(y_hbm.at[ni2], ny2, nsem)]
        else:
            nxt = []
        for cp in pending:
            cp.wait()
        pending = nxt
        y1c, y2c = (y1a, y2a) if even else (y1b, y2b)

        def trow(t, carry):
            wa = w1_v[h * CHT + t]
            wb = w2_v[h * CHT + t]

            def tcol(c, carry2):
                for u in range(4):
                    sl = pl.ds(c * 64 + u * 16, 16)
                    o_v[t, sl] = wa * y1c[t, sl] + wb * y2c[t, sl]
                return carry2

            lax.fori_loop(0, C // 64, tcol, 0)
            return carry

        lax.fori_loop(0, CHT, trow, 0)
        pltpu.sync_copy(o_v, out_hbm.at[pl.ds(base, CHT)])


def _combine_call(y, s1, s2, w1r, w2r):
    fn = functools.partial(
        pl.kernel,
        mesh=plsc.VectorSubcoreMesh(core_axis_name="c", subcore_axis_name="s"),
        out_type=jax.ShapeDtypeStruct((T, C), jnp.float32),
        scratch_types=[
            pltpu.VMEM((CHT,), jnp.int32),
            pltpu.VMEM((CHT,), jnp.int32),
            pltpu.VMEM((CHT,), jnp.int32),
            pltpu.VMEM((CHT,), jnp.int32),
            pltpu.VMEM((CHT, C), jnp.float32),
            pltpu.VMEM((CHT, C), jnp.float32),
            pltpu.VMEM((CHT, C), jnp.float32),
            pltpu.VMEM((CHT, C), jnp.float32),
            pltpu.VMEM((TPW, 16), jnp.float32),
            pltpu.VMEM((TPW, 16), jnp.float32),
            pltpu.VMEM((CHT, C), jnp.float32),
            pltpu.SemaphoreType.DMA,
            pltpu.SemaphoreType.DMA,
        ],
    )(_combine_body)
    return fn(y, s1, s2, w1r, w2r)


def kernel(x, RW1, Rb1, RW2, Rb2, RW3, Rb3, EW1, Eb1, EW2, Eb2):
    probs, w1r, w2r, div, s1, s2, cnt, off = _router_call(
        x, RW1, Rb1, RW2, Rb2, RW3, Rb3)
    s1f = s1.reshape(T)
    s2f = s2.reshape(T)
    xs = _dispatch_call(x, s1f, s2f)
    y = _gmm_call(cnt, off, xs, EW1, Eb1, EW2, Eb2)
    out = _combine_call(y, s1f, s2f, w1r, w2r)
    return out, probs, jnp.float32(0.0), div[0, 0]


# speedup vs baseline: 1.2086x; 1.2086x over previous
import functools

import jax
import jax.numpy as jnp
from jax import lax
from jax.experimental import pallas as pl
from jax.experimental.pallas import tpu as pltpu
from jax.experimental.pallas import tpu_sc as plsc

T, D, HR, E, DFF, C = 2048, 1024, 512, 8, 4096, 1024
BM_R = 512
BM = 256
BF = 1024
F = DFF // BF
M_PAD = 2 * T + E * BM
MAX_ROWS = T
NC, NS, NW = 2, 16, 32
TPW = T // NW
HALF = TPW // 2


def _router_body(x_ref, rw1_ref, rb1_ref, rw2_ref, rb2_ref, rw3_ref, rb3_ref,
                 probs_ref, w1_ref, w2_ref, div_ref, s1_ref, s2_ref, cnt_ref,
                 off_ref, i1sc, i2sc):
    m = pl.program_id(0)
    h1 = jnp.maximum(
        jnp.dot(x_ref[...], rw1_ref[...], preferred_element_type=jnp.float32)
        + rb1_ref[...], 0.0)
    h2 = jnp.maximum(
        jnp.dot(h1, rw2_ref[...], preferred_element_type=jnp.float32)
        + rb2_ref[...], 0.0)
    scores = (jnp.dot(h2, rw3_ref[...], preferred_element_type=jnp.float32)
              + rb3_ref[...])
    mx = jnp.max(scores, axis=1, keepdims=True)
    ex = jnp.exp(scores - mx)
    probs = ex / jnp.sum(ex, axis=1, keepdims=True)
    probs_ref[...] = probs
    iota = jax.lax.broadcasted_iota(jnp.int32, probs.shape, 1)
    p1 = jnp.max(probs, axis=1, keepdims=True)
    i1 = jnp.min(jnp.where(probs == p1, iota, E), axis=1, keepdims=True)
    masked = jnp.where(iota == i1, -jnp.inf, probs)
    p2 = jnp.max(masked, axis=1, keepdims=True)
    i2 = jnp.min(jnp.where(masked == p2, iota, E), axis=1, keepdims=True)
    w1_ref[...] = jnp.broadcast_to(p1 * 0.5, (BM_R, 16))
    w2_ref[...] = jnp.broadcast_to(p2 * 0.5, (BM_R, 16))
    i1sc[pl.ds(pl.multiple_of(m * BM_R, BM_R), BM_R), :] = i1
    i2sc[pl.ds(pl.multiple_of(m * BM_R, BM_R), BM_R), :] = i2
    dv = jnp.sum(scores * scores)

    @pl.when(m == 0)
    def _():
        div_ref[0, 0] = dv

    @pl.when(m != 0)
    def _():
        div_ref[0, 0] += dv

    @pl.when(m == T // BM_R - 1)
    def _():
        ee = jax.lax.broadcasted_iota(jnp.int32, (T, E), 1)
        a1 = (i1sc[...] == ee).astype(jnp.float32)
        a2 = (i2sc[...] == ee).astype(jnp.float32)
        a = a1 + a2
        CH = 512
        ri = jax.lax.broadcasted_iota(jnp.int32, (CH, CH), 0)
        ci = jax.lax.broadcasted_iota(jnp.int32, (CH, CH), 1)
        tri = (ci <= ri).astype(jnp.float32)
        base = jnp.zeros((1, E), jnp.float32)
        chunks = []
        for k in range(T // CH):
            ak = a[k * CH:(k + 1) * CH]
            incl = jnp.dot(tri, ak, preferred_element_type=jnp.float32)
            chunks.append(incl - ak + base)
            base = base + incl[CH - 1:CH, :]
        excl = jnp.concatenate(chunks, axis=0)
        counts = base
        ntiles = jnp.floor((counts + (BM - 1)) * (1.0 / BM))
        padded = ntiles * BM
        ii = jax.lax.broadcasted_iota(jnp.int32, (E, E), 0)
        jj = jax.lax.broadcasted_iota(jnp.int32, (E, E), 1)
        pmat = jnp.broadcast_to(padded, (E, E))
        off_col = jnp.sum(jnp.where(jj < ii, pmat, 0.0), axis=1,
                          keepdims=True)
        cnt_col = jnp.sum(jnp.where(jj == ii,
                                    jnp.broadcast_to(counts, (E, E)), 0.0),
                          axis=1, keepdims=True)
        off_ref[...] = off_col.astype(jnp.int32)
        cnt_ref[...] = cnt_col.astype(jnp.int32)
        rank1 = jnp.sum(excl * a1, axis=1, keepdims=True)
        rank2 = jnp.sum(excl * a2, axis=1, keepdims=True)
        off1 = jnp.dot(a1, off_col, preferred_element_type=jnp.float32)
        off2 = jnp.dot(a2, off_col, preferred_element_type=jnp.float32)
        s1_ref[...] = (off1 + rank1).astype(jnp.int32)
        s2_ref[...] = (off2 + rank2).astype(jnp.int32)


def _router_call(x, RW1, Rb1, RW2, Rb2, RW3, Rb3):
    return pl.pallas_call(
        _router_body,
        grid=(T // BM_R,),
        in_specs=[
            pl.BlockSpec((BM_R, D), lambda m: (m, 0)),
            pl.BlockSpec((D, HR), lambda m: (0, 0)),
            pl.BlockSpec((1, HR), lambda m: (0, 0)),
            pl.BlockSpec((HR, HR // 2), lambda m: (0, 0)),
            pl.BlockSpec((1, HR // 2), lambda m: (0, 0)),
            pl.BlockSpec((HR // 2, E), lambda m: (0, 0)),
            pl.BlockSpec((1, E), lambda m: (0, 0)),
        ],
        out_specs=[
            pl.BlockSpec((BM_R, E), lambda m: (m, 0)),
            pl.BlockSpec((BM_R, 16), lambda m: (m, 0)),
            pl.BlockSpec((BM_R, 16), lambda m: (m, 0)),
            pl.BlockSpec(memory_space=pltpu.SMEM),
            pl.BlockSpec((T, 1), lambda m: (0, 0)),
            pl.BlockSpec((T, 1), lambda m: (0, 0)),
            pl.BlockSpec((E, 1), lambda m: (0, 0)),
            pl.BlockSpec((E, 1), lambda m: (0, 0)),
        ],
        out_shape=[
            jax.ShapeDtypeStruct((T, E), jnp.float32),
            jax.ShapeDtypeStruct((T, 16), jnp.float32),
            jax.ShapeDtypeStruct((T, 16), jnp.float32),
            jax.ShapeDtypeStruct((1, 1), jnp.float32),
            jax.ShapeDtypeStruct((T, 1), jnp.int32),
            jax.ShapeDtypeStruct((T, 1), jnp.int32),
            jax.ShapeDtypeStruct((E, 1), jnp.int32),
            jax.ShapeDtypeStruct((E, 1), jnp.int32),
        ],
        scratch_shapes=[
            pltpu.VMEM((T, 1), jnp.int32),
            pltpu.VMEM((T, 1), jnp.int32),
        ],
        compiler_params=pltpu.CompilerParams(
            dimension_semantics=("arbitrary",)),
    )(x, RW1, Rb1.reshape(1, -1), RW2, Rb2.reshape(1, -1), RW3,
      Rb3.reshape(1, -1))


def _dispatch_body(x_hbm, s1_hbm, s2_hbm, xs_hbm, idx1_v, idx2_v, rows_v,
                   sem):
    wid = lax.axis_index("s") * NC + lax.axis_index("c")
    base = wid * TPW
    pltpu.sync_copy(s1_hbm.at[pl.ds(base, TPW)], idx1_v)
    pltpu.sync_copy(s2_hbm.at[pl.ds(base, TPW)], idx2_v)
    pltpu.sync_copy(x_hbm.at[pl.ds(base, TPW)], rows_v)
    cp1 = pltpu.async_copy(rows_v, xs_hbm.at[idx1_v], sem)
    cp2 = pltpu.async_copy(rows_v, xs_hbm.at[idx2_v], sem)
    cp1.wait()
    cp2.wait()


def _dispatch_call(x, s1, s2):
    fn = functools.partial(
        pl.kernel,
        mesh=plsc.VectorSubcoreMesh(core_axis_name="c", subcore_axis_name="s"),
        out_type=jax.ShapeDtypeStruct((M_PAD, D), jnp.float32),
        scratch_types=[
            pltpu.VMEM((TPW,), jnp.int32),
            pltpu.VMEM((TPW,), jnp.int32),
            pltpu.VMEM((TPW, D), jnp.float32),
            pltpu.SemaphoreType.DMA,
        ],
    )(_dispatch_body)
    return fn(x, s1, s2)


def _gmm_body(cnt_ref, off_ref, xs_ref, ew1_ref, eb1_ref, ew2_ref, eb2_ref,
              y_ref, xsc0, xsc1, ysc0, ysc1, sem1, sem2):
    e = pl.program_id(0)
    f = pl.program_id(1)
    cnt = cnt_ref[e, 0]
    off = off_ref[e, 0]
    nt = (cnt + (BM - 1)) // BM

    def nt_of(ei):
        return (cnt_ref[ei, 0] + (BM - 1)) // BM

    def x_copy(ei, i, buf):
        return pltpu.make_async_copy(
            xs_ref.at[pl.ds(pl.multiple_of(off_ref[ei, 0] + i * BM, BM), BM)],
            buf.at[pl.ds(i * BM, BM)], sem1)

    def y_copy(ei, i, ybuf):
        return pltpu.make_async_copy(
            ybuf.at[pl.ds(i * BM, BM)],
            y_ref.at[pl.ds(pl.multiple_of(off_ref[ei, 0] + i * BM, BM), BM)],
            sem2)

    @pl.when(f == 0)
    def _():
        @pl.when(e == 0)
        def _():
            lax.fori_loop(
                0, nt, lambda i, c: (x_copy(0, i, xsc0).start(), c)[1], 0)

        @pl.when(e % 2 == 0)
        def _():
            lax.fori_loop(
                0, nt, lambda i, c: (x_copy(e, i, xsc0).wait(), c)[1], 0)

        @pl.when(e % 2 == 1)
        def _():
            lax.fori_loop(
                0, nt, lambda i, c: (x_copy(e, i, xsc1).wait(), c)[1], 0)

        @pl.when((e > 1) & (e % 2 == 0))
        def _():
            lax.fori_loop(
                0, nt_of(e - 2),
                lambda i, c: (y_copy(e - 2, i, ysc0).wait(), c)[1], 0)

        @pl.when((e > 1) & (e % 2 == 1))
        def _():
            lax.fori_loop(
                0, nt_of(e - 2),
                lambda i, c: (y_copy(e - 2, i, ysc1).wait(), c)[1], 0)

    @pl.when((f == F - 1) & (e + 1 < E) & (e % 2 == 0))
    def _():
        lax.fori_loop(
            0, nt_of(e + 1),
            lambda i, c: (x_copy(e + 1, i, xsc1).start(), c)[1], 0)

    @pl.when((f == F - 1) & (e + 1 < E) & (e % 2 == 1))
    def _():
        lax.fori_loop(
            0, nt_of(e + 1),
            lambda i, c: (x_copy(e + 1, i, xsc0).start(), c)[1], 0)

    ew1 = ew1_ref[0]
    eb1 = eb1_ref[0]
    ew2 = ew2_ref[0]
    eb2 = eb2_ref[0]

    def tile_loop(xbuf, ybuf):
        def tile(i, carry):
            sl = pl.ds(i * BM, BM)
            hh = jnp.maximum(
                jnp.dot(xbuf[sl, :], ew1, preferred_element_type=jnp.float32)
                + eb1, 0.0)
            contrib = jnp.dot(hh, ew2, preferred_element_type=jnp.float32)

            @pl.when(f == 0)
            def _():
                ybuf[sl, :] = contrib + eb2

            @pl.when(f != 0)
            def _():
                ybuf[sl, :] = ybuf[sl, :] + contrib

            return carry

        lax.fori_loop(0, nt, tile, 0)

    @pl.when(e % 2 == 0)
    def _():
        tile_loop(xsc0, ysc0)

    @pl.when(e % 2 == 1)
    def _():
        tile_loop(xsc1, ysc1)

    @pl.when(f == F - 1)
    def _():
        @pl.when(e % 2 == 0)
        def _():
            lax.fori_loop(
                0, nt, lambda i, c: (y_copy(e, i, ysc0).start(), c)[1], 0)

        @pl.when(e % 2 == 1)
        def _():
            lax.fori_loop(
                0, nt, lambda i, c: (y_copy(e, i, ysc1).start(), c)[1], 0)

        @pl.when(e == E - 1)
        def _():
            lax.fori_loop(
                0, nt_of(E - 2),
                lambda i, c: (y_copy(E - 2, i, ysc0).wait(), c)[1], 0)
            lax.fori_loop(
                0, nt, lambda i, c: (y_copy(e, i, ysc1).wait(), c)[1], 0)


def _gmm_call(cnt, off, xs, EW1, Eb1, EW2, Eb2):
    return pl.pallas_call(
        _gmm_body,
        grid=(E, F),
        in_specs=[
            pl.BlockSpec(memory_space=pltpu.SMEM),
            pl.BlockSpec(memory_space=pltpu.SMEM),
            pl.BlockSpec(memory_space=pl.ANY),
            pl.BlockSpec((1, D, BF), lambda e, f: (e, 0, f)),
            pl.BlockSpec((1, 1, BF), lambda e, f: (e, 0, f)),
            pl.BlockSpec((1, BF, C), lambda e, f: (e, f, 0)),
            pl.BlockSpec((1, 1, C), lambda e, f: (e, 0, 0)),
        ],
        out_specs=pl.BlockSpec(memory_space=pl.ANY),
        out_shape=jax.ShapeDtypeStruct((M_PAD, C), jnp.float32),
        scratch_shapes=[
            pltpu.VMEM((MAX_ROWS, D), jnp.float32),
            pltpu.VMEM((MAX_ROWS, D), jnp.float32),
            pltpu.VMEM((MAX_ROWS, C), jnp.float32),
            pltpu.VMEM((MAX_ROWS, C), jnp.float32),
            pltpu.SemaphoreType.DMA,
            pltpu.SemaphoreType.DMA,
        ],
        compiler_params=pltpu.CompilerParams(
            dimension_semantics=("arbitrary", "arbitrary")),
    )(cnt, off, xs, EW1, Eb1.reshape(E, 1, DFF), EW2, Eb2.reshape(E, 1, C))


CHT = 16
NH = TPW // CHT


def _combine_body(y_hbm, s1_hbm, s2_hbm, w1_hbm, w2_hbm, out_hbm,
                  ia1, ia2, ib1, ib2, y1a, y2a, y1b, y2b, w1_v, w2_v, o_v,
                  sema, semb):
    wid = lax.axis_index("s") * NC + lax.axis_index("c")
    tbase = wid * TPW
    pltpu.sync_copy(w1_hbm.at[pl.ds(tbase, TPW)], w1_v)
    pltpu.sync_copy(w2_hbm.at[pl.ds(tbase, TPW)], w2_v)
    pltpu.sync_copy(s1_hbm.at[pl.ds(tbase, CHT)], ia1)
    pltpu.sync_copy(s2_hbm.at[pl.ds(tbase, CHT)], ia2)
    pending = [pltpu.async_copy(y_hbm.at[ia1], y1a, sema),
               pltpu.async_copy(y_hbm.at[ia2], y2a, sema)]
    for h in range(NH):
        base = tbase + h * CHT
        even = (h % 2 == 0)
        if h + 1 < NH:
            nbase = base + CHT
            (ni1, ni2, ny1, ny2, nsem) = (
                (ib1, ib2, y1b, y2b, semb) if even else
                (ia1, ia2, y1a, y2a, sema))
            pltpu.sync_copy(s1_hbm.at[pl.ds(nbase, CHT)], ni1)
            pltpu.sync_copy(s2_hbm.at[pl.ds(nbase, CHT)], ni2)
            nxt = [pltpu.async_copy(y_hbm.at[ni1], ny1, nsem),
                   pltpu.async_copy(y_hbm.at[ni2], ny2, nsem)]
        else:
            nxt = []
        for cp in pending:
            cp.wait()
        pending = nxt
        y1c, y2c = (y1a, y2a) if even else (y1b, y2b)

        def trow(t, carry):
            wa = w1_v[h * CHT + t]
            wb = w2_v[h * CHT + t]

            def tcol(c, carry2):
                for u in range(4):
                    sl = pl.ds(c * 64 + u * 16, 16)
                    o_v[t, sl] = wa * y1c[t, sl] + wb * y2c[t, sl]
                return carry2

            lax.fori_loop(0, C // 64, tcol, 0)
            return carry

        lax.fori_loop(0, CHT, trow, 0)
        pltpu.sync_copy(o_v, out_hbm.at[pl.ds(base, CHT)])


def _combine_call(y, s1, s2, w1r, w2r):
    fn = functools.partial(
        pl.kernel,
        mesh=plsc.VectorSubcoreMesh(core_axis_name="c", subcore_axis_name="s"),
        out_type=jax.ShapeDtypeStruct((T, C), jnp.float32),
        scratch_types=[
            pltpu.VMEM((CHT,), jnp.int32),
            pltpu.VMEM((CHT,), jnp.int32),
            pltpu.VMEM((CHT,), jnp.int32),
            pltpu.VMEM((CHT,), jnp.int32),
            pltpu.VMEM((CHT, C), jnp.float32),
            pltpu.VMEM((CHT, C), jnp.float32),
            pltpu.VMEM((CHT, C), jnp.float32),
            pltpu.VMEM((CHT, C), jnp.float32),
            pltpu.VMEM((TPW, 16), jnp.float32),
            pltpu.VMEM((TPW, 16), jnp.float32),
            pltpu.VMEM((CHT, C), jnp.float32),
            pltpu.SemaphoreType.DMA,
            pltpu.SemaphoreType.DMA,
        ],
    )(_combine_body)
    return fn(y, s1, s2, w1r, w2r)


def kernel(x, RW1, Rb1, RW2, Rb2, RW3, Rb3, EW1, Eb1, EW2, Eb2):
    probs, w1r, w2r, div, s1, s2, cnt, off = _router_call(
        x, RW1, Rb1, RW2, Rb2, RW3, Rb3)
    s1f = s1.reshape(T)
    s2f = s2.reshape(T)
    xs = _dispatch_call(x, s1f, s2f)
    y = _gmm_call(cnt, off, xs, EW1, Eb1, EW2, Eb2)
    out = _combine_call(y, s1f, s2f, w1r, w2r)
    return out, probs, jnp.float32(0.0), div[0, 0]


# speedup vs baseline: 1.2131x; 1.0038x over previous
import functools

import jax
import jax.numpy as jnp
from jax import lax
from jax.experimental import pallas as pl
from jax.experimental.pallas import tpu as pltpu
from jax.experimental.pallas import tpu_sc as plsc

T, D, HR, E, DFF, C = 2048, 1024, 512, 8, 4096, 1024
BM_R = 512
BM = 256
BF = 1024
F = DFF // BF
M_PAD = 2 * T + E * BM
MAX_ROWS = T
NC, NS, NW = 2, 16, 32
TPW = T // NW
HALF = TPW // 2


def _router_body(x_ref, rw1_ref, rb1_ref, rw2_ref, rb2_ref, rw3_ref, rb3_ref,
                 probs_ref, w1_ref, w2_ref, div_ref, s1_ref, s2_ref, cnt_ref,
                 off_ref, i1sc, i2sc):
    m = pl.program_id(0)
    h1 = jnp.maximum(
        jnp.dot(x_ref[...], rw1_ref[...], preferred_element_type=jnp.float32)
        + rb1_ref[...], 0.0)
    h2 = jnp.maximum(
        jnp.dot(h1, rw2_ref[...], preferred_element_type=jnp.float32)
        + rb2_ref[...], 0.0)
    scores = (jnp.dot(h2, rw3_ref[...], preferred_element_type=jnp.float32)
              + rb3_ref[...])
    mx = jnp.max(scores, axis=1, keepdims=True)
    ex = jnp.exp(scores - mx)
    probs = ex / jnp.sum(ex, axis=1, keepdims=True)
    probs_ref[...] = probs
    iota = jax.lax.broadcasted_iota(jnp.int32, probs.shape, 1)
    p1 = jnp.max(probs, axis=1, keepdims=True)
    i1 = jnp.min(jnp.where(probs == p1, iota, E), axis=1, keepdims=True)
    masked = jnp.where(iota == i1, -jnp.inf, probs)
    p2 = jnp.max(masked, axis=1, keepdims=True)
    i2 = jnp.min(jnp.where(masked == p2, iota, E), axis=1, keepdims=True)
    w1_ref[...] = jnp.broadcast_to(p1 * 0.5, (BM_R, 16))
    w2_ref[...] = jnp.broadcast_to(p2 * 0.5, (BM_R, 16))
    i1sc[pl.ds(pl.multiple_of(m * BM_R, BM_R), BM_R), :] = i1
    i2sc[pl.ds(pl.multiple_of(m * BM_R, BM_R), BM_R), :] = i2
    dv = jnp.sum(scores * scores)

    @pl.when(m == 0)
    def _():
        div_ref[0, 0] = dv

    @pl.when(m != 0)
    def _():
        div_ref[0, 0] += dv

    @pl.when(m == T // BM_R - 1)
    def _():
        ee = jax.lax.broadcasted_iota(jnp.int32, (T, E), 1)
        a1 = (i1sc[...] == ee).astype(jnp.float32)
        a2 = (i2sc[...] == ee).astype(jnp.float32)
        a = a1 + a2
        CH = 512
        ri = jax.lax.broadcasted_iota(jnp.int32, (CH, CH), 0)
        ci = jax.lax.broadcasted_iota(jnp.int32, (CH, CH), 1)
        tri = (ci <= ri).astype(jnp.float32)
        base = jnp.zeros((1, E), jnp.float32)
        chunks = []
        for k in range(T // CH):
            ak = a[k * CH:(k + 1) * CH]
            incl = jnp.dot(tri, ak, preferred_element_type=jnp.float32)
            chunks.append(incl - ak + base)
            base = base + incl[CH - 1:CH, :]
        excl = jnp.concatenate(chunks, axis=0)
        counts = base
        ntiles = jnp.floor((counts + (BM - 1)) * (1.0 / BM))
        padded = ntiles * BM
        ii = jax.lax.broadcasted_iota(jnp.int32, (E, E), 0)
        jj = jax.lax.broadcasted_iota(jnp.int32, (E, E), 1)
        pmat = jnp.broadcast_to(padded, (E, E))
        off_col = jnp.sum(jnp.where(jj < ii, pmat, 0.0), axis=1,
                          keepdims=True)
        cnt_col = jnp.sum(jnp.where(jj == ii,
                                    jnp.broadcast_to(counts, (E, E)), 0.0),
                          axis=1, keepdims=True)
        off_ref[...] = off_col.astype(jnp.int32)
        cnt_ref[...] = cnt_col.astype(jnp.int32)
        rank1 = jnp.sum(excl * a1, axis=1, keepdims=True)
        rank2 = jnp.sum(excl * a2, axis=1, keepdims=True)
        off1 = jnp.dot(a1, off_col, preferred_element_type=jnp.float32)
        off2 = jnp.dot(a2, off_col, preferred_element_type=jnp.float32)
        s1_ref[...] = (off1 + rank1).astype(jnp.int32)
        s2_ref[...] = (off2 + rank2).astype(jnp.int32)


def _router_call(x, RW1, Rb1, RW2, Rb2, RW3, Rb3):
    return pl.pallas_call(
        _router_body,
        grid=(T // BM_R,),
        in_specs=[
            pl.BlockSpec((BM_R, D), lambda m: (m, 0)),
            pl.BlockSpec((D, HR), lambda m: (0, 0)),
            pl.BlockSpec((1, HR), lambda m: (0, 0)),
            pl.BlockSpec((HR, HR // 2), lambda m: (0, 0)),
            pl.BlockSpec((1, HR // 2), lambda m: (0, 0)),
            pl.BlockSpec((HR // 2, E), lambda m: (0, 0)),
            pl.BlockSpec((1, E), lambda m: (0, 0)),
        ],
        out_specs=[
            pl.BlockSpec((BM_R, E), lambda m: (m, 0)),
            pl.BlockSpec((BM_R, 16), lambda m: (m, 0)),
            pl.BlockSpec((BM_R, 16), lambda m: (m, 0)),
            pl.BlockSpec(memory_space=pltpu.SMEM),
            pl.BlockSpec((T, 1), lambda m: (0, 0)),
            pl.BlockSpec((T, 1), lambda m: (0, 0)),
            pl.BlockSpec((E, 1), lambda m: (0, 0)),
            pl.BlockSpec((E, 1), lambda m: (0, 0)),
        ],
        out_shape=[
            jax.ShapeDtypeStruct((T, E), jnp.float32),
            jax.ShapeDtypeStruct((T, 16), jnp.float32),
            jax.ShapeDtypeStruct((T, 16), jnp.float32),
            jax.ShapeDtypeStruct((1, 1), jnp.float32),
            jax.ShapeDtypeStruct((T, 1), jnp.int32),
            jax.ShapeDtypeStruct((T, 1), jnp.int32),
            jax.ShapeDtypeStruct((E, 1), jnp.int32),
            jax.ShapeDtypeStruct((E, 1), jnp.int32),
        ],
        scratch_shapes=[
            pltpu.VMEM((T, 1), jnp.int32),
            pltpu.VMEM((T, 1), jnp.int32),
        ],
        compiler_params=pltpu.CompilerParams(
            dimension_semantics=("arbitrary",)),
    )(x, RW1, Rb1.reshape(1, -1), RW2, Rb2.reshape(1, -1), RW3,
      Rb3.reshape(1, -1))


def _dispatch_body(x_hbm, s1_hbm, s2_hbm, xs_hbm, idx1_v, idx2_v, rows_v,
                   sem):
    wid = lax.axis_index("s") * NC + lax.axis_index("c")
    base = wid * TPW
    ld1 = pltpu.async_copy(s1_hbm.at[pl.ds(base, TPW)], idx1_v, sem)
    ld2 = pltpu.async_copy(s2_hbm.at[pl.ds(base, TPW)], idx2_v, sem)
    ldx = pltpu.async_copy(x_hbm.at[pl.ds(base, TPW)], rows_v, sem)
    ld1.wait()
    ld2.wait()
    ldx.wait()
    cp1 = pltpu.async_copy(rows_v, xs_hbm.at[idx1_v], sem)
    cp2 = pltpu.async_copy(rows_v, xs_hbm.at[idx2_v], sem)
    cp1.wait()
    cp2.wait()


def _dispatch_call(x, s1, s2):
    fn = functools.partial(
        pl.kernel,
        mesh=plsc.VectorSubcoreMesh(core_axis_name="c", subcore_axis_name="s"),
        out_type=jax.ShapeDtypeStruct((M_PAD, D), jnp.float32),
        scratch_types=[
            pltpu.VMEM((TPW,), jnp.int32),
            pltpu.VMEM((TPW,), jnp.int32),
            pltpu.VMEM((TPW, D), jnp.float32),
            pltpu.SemaphoreType.DMA,
        ],
    )(_dispatch_body)
    return fn(x, s1, s2)


def _gmm_body(cnt_ref, off_ref, xs_ref, ew1_ref, eb1_ref, ew2_ref, eb2_ref,
              y_ref, xsc0, xsc1, ysc0, ysc1, sem1, sem2):
    e = pl.program_id(0)
    f = pl.program_id(1)
    cnt = cnt_ref[e, 0]
    off = off_ref[e, 0]
    nt = (cnt + (BM - 1)) // BM

    def nt_of(ei):
        return (cnt_ref[ei, 0] + (BM - 1)) // BM

    def x_copy(ei, i, buf):
        return pltpu.make_async_copy(
            xs_ref.at[pl.ds(pl.multiple_of(off_ref[ei, 0] + i * BM, BM), BM)],
            buf.at[pl.ds(i * BM, BM)], sem1)

    def y_copy(ei, i, ybuf):
        return pltpu.make_async_copy(
            ybuf.at[pl.ds(i * BM, BM)],
            y_ref.at[pl.ds(pl.multiple_of(off_ref[ei, 0] + i * BM, BM), BM)],
            sem2)

    @pl.when(f == 0)
    def _():
        @pl.when(e == 0)
        def _():
            lax.fori_loop(
                0, nt, lambda i, c: (x_copy(0, i, xsc0).start(), c)[1], 0)

        @pl.when(e % 2 == 0)
        def _():
            lax.fori_loop(
                0, nt, lambda i, c: (x_copy(e, i, xsc0).wait(), c)[1], 0)

        @pl.when(e % 2 == 1)
        def _():
            lax.fori_loop(
                0, nt, lambda i, c: (x_copy(e, i, xsc1).wait(), c)[1], 0)

        @pl.when((e > 1) & (e % 2 == 0))
        def _():
            lax.fori_loop(
                0, nt_of(e - 2),
                lambda i, c: (y_copy(e - 2, i, ysc0).wait(), c)[1], 0)

        @pl.when((e > 1) & (e % 2 == 1))
        def _():
            lax.fori_loop(
                0, nt_of(e - 2),
                lambda i, c: (y_copy(e - 2, i, ysc1).wait(), c)[1], 0)

    @pl.when((f == F - 1) & (e + 1 < E) & (e % 2 == 0))
    def _():
        lax.fori_loop(
            0, nt_of(e + 1),
            lambda i, c: (x_copy(e + 1, i, xsc1).start(), c)[1], 0)

    @pl.when((f == F - 1) & (e + 1 < E) & (e % 2 == 1))
    def _():
        lax.fori_loop(
            0, nt_of(e + 1),
            lambda i, c: (x_copy(e + 1, i, xsc0).start(), c)[1], 0)

    ew1 = ew1_ref[0]
    eb1 = eb1_ref[0]
    ew2 = ew2_ref[0]
    eb2 = eb2_ref[0]

    def tile_loop(xbuf, ybuf):
        def tile(i, carry):
            sl = pl.ds(i * BM, BM)
            hh = jnp.maximum(
                jnp.dot(xbuf[sl, :], ew1, preferred_element_type=jnp.float32)
                + eb1, 0.0)
            contrib = jnp.dot(hh, ew2, preferred_element_type=jnp.float32)

            @pl.when(f == 0)
            def _():
                ybuf[sl, :] = contrib + eb2

            @pl.when(f != 0)
            def _():
                ybuf[sl, :] = ybuf[sl, :] + contrib

            return carry

        lax.fori_loop(0, nt, tile, 0)

    @pl.when(e % 2 == 0)
    def _():
        tile_loop(xsc0, ysc0)

    @pl.when(e % 2 == 1)
    def _():
        tile_loop(xsc1, ysc1)

    @pl.when(f == F - 1)
    def _():
        @pl.when(e % 2 == 0)
        def _():
            lax.fori_loop(
                0, nt, lambda i, c: (y_copy(e, i, ysc0).start(), c)[1], 0)

        @pl.when(e % 2 == 1)
        def _():
            lax.fori_loop(
                0, nt, lambda i, c: (y_copy(e, i, ysc1).start(), c)[1], 0)

        @pl.when(e == E - 1)
        def _():
            lax.fori_loop(
                0, nt_of(E - 2),
                lambda i, c: (y_copy(E - 2, i, ysc0).wait(), c)[1], 0)
            lax.fori_loop(
                0, nt, lambda i, c: (y_copy(e, i, ysc1).wait(), c)[1], 0)


def _gmm_call(cnt, off, xs, EW1, Eb1, EW2, Eb2):
    return pl.pallas_call(
        _gmm_body,
        grid=(E, F),
        in_specs=[
            pl.BlockSpec(memory_space=pltpu.SMEM),
            pl.BlockSpec(memory_space=pltpu.SMEM),
            pl.BlockSpec(memory_space=pl.ANY),
            pl.BlockSpec((1, D, BF), lambda e, f: (e, 0, f)),
            pl.BlockSpec((1, 1, BF), lambda e, f: (e, 0, f)),
            pl.BlockSpec((1, BF, C), lambda e, f: (e, f, 0)),
            pl.BlockSpec((1, 1, C), lambda e, f: (e, 0, 0)),
        ],
        out_specs=pl.BlockSpec(memory_space=pl.ANY),
        out_shape=jax.ShapeDtypeStruct((M_PAD, C), jnp.float32),
        scratch_shapes=[
            pltpu.VMEM((MAX_ROWS, D), jnp.float32),
            pltpu.VMEM((MAX_ROWS, D), jnp.float32),
            pltpu.VMEM((MAX_ROWS, C), jnp.float32),
            pltpu.VMEM((MAX_ROWS, C), jnp.float32),
            pltpu.SemaphoreType.DMA,
            pltpu.SemaphoreType.DMA,
        ],
        compiler_params=pltpu.CompilerParams(
            dimension_semantics=("arbitrary", "arbitrary")),
    )(cnt, off, xs, EW1, Eb1.reshape(E, 1, DFF), EW2, Eb2.reshape(E, 1, C))


CHT = 16
NH = TPW // CHT


def _combine_body(y_hbm, s1_hbm, s2_hbm, w1_hbm, w2_hbm, out_hbm,
                  ia1, ia2, ib1, ib2, y1a, y2a, y1b, y2b, w1_v, w2_v, o_v,
                  sema, semb):
    wid = lax.axis_index("s") * NC + lax.axis_index("c")
    tbase = wid * TPW
    pltpu.sync_copy(w1_hbm.at[pl.ds(tbase, TPW)], w1_v)
    pltpu.sync_copy(w2_hbm.at[pl.ds(tbase, TPW)], w2_v)
    pltpu.sync_copy(s1_hbm.at[pl.ds(tbase, CHT)], ia1)
    pltpu.sync_copy(s2_hbm.at[pl.ds(tbase, CHT)], ia2)
    pending = [pltpu.async_copy(y_hbm.at[ia1], y1a, sema),
               pltpu.async_copy(y_hbm.at[ia2], y2a, sema)]
    for h in range(NH):
        base = tbase + h * CHT
        even = (h % 2 == 0)
        if h + 1 < NH:
            nbase = base + CHT
            (ni1, ni2, ny1, ny2, nsem) = (
                (ib1, ib2, y1b, y2b, semb) if even else
                (ia1, ia2, y1a, y2a, sema))
            pltpu.sync_copy(s1_hbm.at[pl.ds(nbase, CHT)], ni1)
            pltpu.sync_copy(s2_hbm.at[pl.ds(nbase, CHT)], ni2)
            nxt = [pltpu.async_copy(y_hbm.at[ni1], ny1, nsem),
                   pltpu.async_copy(y_hbm.at[ni2], ny2, nsem)]
        else:
            nxt = []
        for cp in pending:
            cp.wait()
        pending = nxt
        y1c, y2c = (y1a, y2a) if even else (y1b, y2b)

        def trow(t, carry):
            wa = w1_v[h * CHT + t]
            wb = w2_v[h * CHT + t]

            def tcol(c, carry2):
                for u in range(4):
                    sl = pl.ds(c * 64 + u * 16, 16)
                    o_v[t, sl] = wa * y1c[t, sl] + wb * y2c[t, sl]
                return carry2

            lax.fori_loop(0, C // 64, tcol, 0)
            return carry

        lax.fori_loop(0, CHT, trow, 0)
        pltpu.sync_copy(o_v, out_hbm.at[pl.ds(base, CHT)])


def _combine_call(y, s1, s2, w1r, w2r):
    fn = functools.partial(
        pl.kernel,
        mesh=plsc.VectorSubcoreMesh(core_axis_name="c", subcore_axis_name="s"),
        out_type=jax.ShapeDtypeStruct((T, C), jnp.float32),
        scratch_types=[
            pltpu.VMEM((CHT,), jnp.int32),
            pltpu.VMEM((CHT,), jnp.int32),
            pltpu.VMEM((CHT,), jnp.int32),
            pltpu.VMEM((CHT,), jnp.int32),
            pltpu.VMEM((CHT, C), jnp.float32),
            pltpu.VMEM((CHT, C), jnp.float32),
            pltpu.VMEM((CHT, C), jnp.float32),
            pltpu.VMEM((CHT, C), jnp.float32),
            pltpu.VMEM((TPW, 16), jnp.float32),
            pltpu.VMEM((TPW, 16), jnp.float32),
            pltpu.VMEM((CHT, C), jnp.float32),
            pltpu.SemaphoreType.DMA,
            pltpu.SemaphoreType.DMA,
        ],
    )(_combine_body)
    return fn(y, s1, s2, w1r, w2r)


def kernel(x, RW1, Rb1, RW2, Rb2, RW3, Rb3, EW1, Eb1, EW2, Eb2):
    probs, w1r, w2r, div, s1, s2, cnt, off = _router_call(
        x, RW1, Rb1, RW2, Rb2, RW3, Rb3)
    s1f = s1.reshape(T)
    s2f = s2.reshape(T)
    xs = _dispatch_call(x, s1f, s2f)
    y = _gmm_call(cnt, off, xs, EW1, Eb1, EW2, Eb2)
    out = _combine_call(y, s1f, s2f, w1r, w2r)
    return out, probs, jnp.float32(0.0), div[0, 0]
